# baseline (speedup 1.0000x reference)
import functools

import jax
import jax.numpy as jnp
from jax import lax
from jax.experimental import pallas as pl
from jax.experimental.pallas import tpu as pltpu
from jax.experimental.pallas import tpu_sc as plsc

N, E, F_IN, D, B = 10000, 160000, 128, 32, 256

NC, NS = 2, 16
NW = NC * NS
CHUNK = 128
CH_PER_W = 40
EP = NW * CH_PER_W * CHUNK
NROW = EP // CHUNK
NPAD = 10048
ROWS_PER_SUB = NPAD // NS

_MESH = dict(core_axis_name="c", subcore_axis_name="s", num_cores=NC,
             num_subcores=NS)
_SC_PARAMS = pltpu.CompilerParams(use_tc_tiling_on_sc=False)


def _sc_gather(table, idx2d):
    mesh = plsc.VectorSubcoreMesh(**_MESH)

    @functools.partial(
        pl.kernel,
        out_type=jax.ShapeDtypeStruct((NROW, CHUNK, D), jnp.float32),
        mesh=mesh,
        scratch_types=[pltpu.VMEM((CHUNK,), jnp.int32),
                       pltpu.VMEM((CHUNK, D), jnp.float32),
                       pltpu.SemaphoreType.DMA],
        compiler_params=_SC_PARAMS,
    )
    def k(table_hbm, idx_hbm, out_hbm, idx_v, rows_v, sem):
        wid = lax.axis_index("s") * NC + lax.axis_index("c")

        def body(j, carry):
            r = wid * CH_PER_W + j
            pltpu.sync_copy(idx_hbm.at[r], idx_v)
            pltpu.async_copy(table_hbm.at[idx_v], rows_v, sem).wait()
            pltpu.sync_copy(rows_v, out_hbm.at[r])
            return carry

        lax.fori_loop(0, CH_PER_W, body, 0)

    return k(table, idx2d)


def _sc_scatter_add(val3, idx2d, zeros):
    mesh = plsc.VectorSubcoreMesh(**_MESH)

    @functools.partial(
        pl.kernel,
        out_type=jax.ShapeDtypeStruct((NC, NPAD, D), jnp.float32),
        mesh=mesh,
        scratch_types=[pltpu.VMEM((CHUNK,), jnp.int32),
                       pltpu.VMEM((CHUNK, D), jnp.float32),
                       pltpu.VMEM_SHARED((NPAD, D), jnp.float32)],
        compiler_params=_SC_PARAMS,
    )
    def k(val_hbm, idx_hbm, z_hbm, out_hbm, idx_v, val_v, acc):
        cid = lax.axis_index("c")
        sid = lax.axis_index("s")
        wid = sid * NC + cid
        rows0 = sid * ROWS_PER_SUB
        pltpu.sync_copy(z_hbm.at[pl.ds(rows0, ROWS_PER_SUB)],
                        acc.at[pl.ds(rows0, ROWS_PER_SUB)])
        plsc.subcore_barrier()

        def body(j, carry):
            r = wid * CH_PER_W + j
            pltpu.sync_copy(idx_hbm.at[r], idx_v)
            pltpu.sync_copy(val_hbm.at[r], val_v)
            pltpu.sync_copy(val_v, acc.at[idx_v], add=True)
            return carry

        lax.fori_loop(0, CH_PER_W, body, 0)
        plsc.subcore_barrier()
        pltpu.sync_copy(acc.at[pl.ds(rows0, ROWS_PER_SUB)],
                        out_hbm.at[cid, pl.ds(rows0, ROWS_PER_SUB)])

    return k(val3, idx2d, zeros)


def _tc_node_init(x, W0, b0):
    def body(x_ref, w_ref, b_ref, o_ref):
        o_ref[...] = jnp.maximum(
            jnp.dot(x_ref[...], w_ref[...],
                    preferred_element_type=jnp.float32) + b_ref[...], 0.0)

    return pl.pallas_call(
        body, out_shape=jax.ShapeDtypeStruct((N, D), jnp.float32),
    )(x, W0, b0.reshape(1, D))


_TILE_H1 = 16384


def _tc_edge_mlp1(ea_p, We1p, be1):
    def body(a_ref, w_ref, b_ref, o_ref):
        o_ref[...] = jnp.maximum(
            jnp.dot(a_ref[...], w_ref[...],
                    preferred_element_type=jnp.float32) + b_ref[...], 0.0)

    return pl.pallas_call(
        body,
        grid=(EP // _TILE_H1,),
        in_specs=[pl.BlockSpec((_TILE_H1, 8), lambda i: (i, 0)),
                  pl.BlockSpec((8, 128), lambda i: (0, 0)),
                  pl.BlockSpec((1, 128), lambda i: (0, 0))],
        out_specs=pl.BlockSpec((_TILE_H1, 128), lambda i: (i, 0)),
        out_shape=jax.ShapeDtypeStruct((EP, 128), jnp.float32),
    )(ea_p, We1p, be1.reshape(1, 128))


_TILE_MSG = 2048


def _tc_msg(h1, g, We2, be2):
    def body(h1_ref, g_ref, w_ref, b_ref, o_ref):
        h2 = jnp.dot(h1_ref[...], w_ref[...],
                     preferred_element_type=jnp.float32) + b_ref[...]
        gv = g_ref[...]
        acc = gv[:, 0:1] * h2[:, 0:D]
        for d in range(1, D):
            acc = acc + gv[:, d:d + 1] * h2[:, d * D:(d + 1) * D]
        o_ref[...] = acc

    return pl.pallas_call(
        body,
        grid=(EP // _TILE_MSG,),
        in_specs=[pl.BlockSpec((_TILE_MSG, 128), lambda i: (i, 0)),
                  pl.BlockSpec((_TILE_MSG, D), lambda i: (i, 0)),
                  pl.BlockSpec((128, D * D), lambda i: (0, 0)),
                  pl.BlockSpec((1, D * D), lambda i: (0, 0))],
        out_specs=pl.BlockSpec((_TILE_MSG, D), lambda i: (i, 0)),
        out_shape=jax.ShapeDtypeStruct((EP, D), jnp.float32),
    )(h1, g, We2, be2.reshape(1, D * D))


def _tc_node_update(a0, a1, d0, d1, st, Wroot, bconv, gru_w):
    (wr, wz, wn, ur, uz, un, br, bz, bn, cr, cz, cn) = gru_w

    def body(a0_ref, a1_ref, d0_ref, d1_ref, s_ref, wroot_ref, bc_ref,
             wr_ref, wz_ref, wn_ref, ur_ref, uz_ref, un_ref,
             br_ref, bz_ref, bn_ref, cr_ref, cz_ref, cn_ref, o_ref):
        deg = jnp.maximum(d0_ref[...] + d1_ref[...], 1.0)
        s = s_ref[...]
        aggr = (a0_ref[...] + a1_ref[...]) / deg
        m = jnp.maximum(
            aggr + jnp.dot(s, wroot_ref[...],
                           preferred_element_type=jnp.float32) + bc_ref[...],
            0.0)
        mm = lambda a, w: jnp.dot(a, w[...], preferred_element_type=jnp.float32)
        r = jax.nn.sigmoid(mm(m, wr_ref) + br_ref[...]
                           + mm(s, ur_ref) + cr_ref[...])
        z = jax.nn.sigmoid(mm(m, wz_ref) + bz_ref[...]
                           + mm(s, uz_ref) + cz_ref[...])
        n = jnp.tanh(mm(m, wn_ref) + bn_ref[...]
                     + r * (mm(s, un_ref) + cn_ref[...]))
        o_ref[...] = (1.0 - z) * n + z * s

    return pl.pallas_call(
        body, out_shape=jax.ShapeDtypeStruct((N, D), jnp.float32),
    )(a0, a1, d0, d1, st, Wroot, bconv.reshape(1, D),
      wr, wz, wn, ur, uz, un,
      br.reshape(1, D), bz.reshape(1, D), bn.reshape(1, D),
      cr.reshape(1, D), cz.reshape(1, D), cn.reshape(1, D))


def _tc_set2set(conv1, st, bcol, fw, Wlin, blin, lstm_w, W1, b1, W2, b2):
    (wi_i, wi_f, wi_g, wi_o, ui_i, ui_f, ui_g, ui_o,
     bl_i, bl_f, bl_g, bl_o) = lstm_w

    def body(c_ref, h_ref, b_ref, fw_ref, wlin_ref, blin_ref,
             wii_ref, wif_ref, wig_ref, wio_ref,
             uii_ref, uif_ref, uig_ref, uio_ref,
             bli_ref, blf_ref, blg_ref, blo_ref,
             w1_ref, b1_ref, w2_ref, b2_ref, o_ref):
        mm = lambda a, w: jnp.dot(a, w, preferred_element_type=jnp.float32)
        conv1_v = c_ref[...]
        out = h_ref[...]
        fwv = fw_ref[...]
        wlin = wlin_ref[...]
        blin = blin_ref[...]
        for _ in range(3):
            out = out + jnp.maximum(mm(fwv * out, wlin) + blin, 0.0)
        xx = jnp.concatenate([conv1_v, out], axis=1)
        bcol_v = b_ref[...]
        iot = lax.broadcasted_iota(jnp.int32, (N, B), 1)
        oh = bcol_v == iot
        ohf = oh.astype(jnp.float32)
        q_star = jnp.zeros((B, 4 * D), jnp.float32)
        hl = jnp.zeros((B, 2 * D), jnp.float32)
        cl = jnp.zeros((B, 2 * D), jnp.float32)
        for _ in range(3):
            gi = mm(q_star, wii_ref[...]) + mm(hl, uii_ref[...]) + bli_ref[...]
            gf = mm(q_star, wif_ref[...]) + mm(hl, uif_ref[...]) + blf_ref[...]
            gg = mm(q_star, wig_ref[...]) + mm(hl, uig_ref[...]) + blg_ref[...]
            go = mm(q_star, wio_ref[...]) + mm(hl, uio_ref[...]) + blo_ref[...]
            cl = jax.nn.sigmoid(gf) * cl + jax.nn.sigmoid(gi) * jnp.tanh(gg)
            hl = jax.nn.sigmoid(go) * jnp.tanh(cl)
            q = hl
            eq = lax.dot_general(xx, q, (((1,), (1,)), ((), ())))
            e = jnp.sum(eq * ohf, axis=1, keepdims=True)
            masked = jnp.where(oh, e, -1e30)
            mseg = jnp.max(masked, axis=0, keepdims=True)
            mseg = jnp.where(mseg > -1e29, mseg, 0.0)
            ex = jnp.exp(e - jnp.sum(ohf * mseg, axis=1, keepdims=True))
            sseg = jnp.sum(ohf * ex, axis=0, keepdims=True)
            a = ex / (jnp.sum(ohf * sseg, axis=1, keepdims=True) + 1e-16)
            rvec = lax.dot_general(ohf, a * xx,
                                   (((0,), (0,)), ((), ())))
            q_star = jnp.concatenate([q, rvec], axis=1)
        o1 = jnp.maximum(mm(q_star, w1_ref[...]) + b1_ref[...], 0.0)
        o_ref[...] = mm(o1, w2_ref[...]) + b2_ref[...]

    return pl.pallas_call(
        body, out_shape=jax.ShapeDtypeStruct((B, 1), jnp.float32),
    )(conv1, st, bcol, fw, Wlin, blin.reshape(1, D),
      wi_i, wi_f, wi_g, wi_o, ui_i, ui_f, ui_g, ui_o,
      bl_i.reshape(1, 2 * D), bl_f.reshape(1, 2 * D),
      bl_g.reshape(1, 2 * D), bl_o.reshape(1, 2 * D),
      W1, b1.reshape(1, 4 * D), W2, b2.reshape(1, 1))


def kernel(x, edge_index, edge_attr, batch, W0, b0, We1, be1, We2, be2,
           Wroot, bconv, Wih, bih, Whh, bhh, fw, Wlin, blin,
           Wih_l, bih_l, Whh_l, bhh_l, W1, b1, W2, b2):
    src = jnp.pad(edge_index[0], (0, EP - E)).reshape(NROW, CHUNK)
    dst = jnp.pad(edge_index[1], (0, EP - E),
                  constant_values=N).reshape(NROW, CHUNK)
    ea_p = jnp.pad(edge_attr, ((0, EP - E), (0, 3)))
    We1p = jnp.pad(We1, ((0, 3), (0, 0)))
    zeros = jnp.zeros((NPAD, D), jnp.float32)
    ones3 = jnp.ones((NROW, CHUNK, D), jnp.float32)

    gru_w = (Wih[:, 0:D], Wih[:, D:2 * D], Wih[:, 2 * D:],
             Whh[:, 0:D], Whh[:, D:2 * D], Whh[:, 2 * D:],
             bih[0:D], bih[D:2 * D], bih[2 * D:],
             bhh[0:D], bhh[D:2 * D], bhh[2 * D:])
    Hs = 2 * D
    lstm_w = (Wih_l[:, 0:Hs], Wih_l[:, Hs:2 * Hs],
              Wih_l[:, 2 * Hs:3 * Hs], Wih_l[:, 3 * Hs:],
              Whh_l[:, 0:Hs], Whh_l[:, Hs:2 * Hs],
              Whh_l[:, 2 * Hs:3 * Hs], Whh_l[:, 3 * Hs:],
              bih_l[0:Hs] + bhh_l[0:Hs], bih_l[Hs:2 * Hs] + bhh_l[Hs:2 * Hs],
              bih_l[2 * Hs:3 * Hs] + bhh_l[2 * Hs:3 * Hs],
              bih_l[3 * Hs:] + bhh_l[3 * Hs:])

    st = _tc_node_init(x, W0, b0)
    h1 = _tc_edge_mlp1(ea_p, We1p, be1)

    deg_p = _sc_scatter_add(ones3, dst, zeros)
    d0 = deg_p[0, :N, 0:1]
    d1 = deg_p[1, :N, 0:1]

    for _ in range(3):
        g3 = _sc_gather(st, src)
        msg = _tc_msg(h1, g3.reshape(EP, D), We2, be2)
        part = _sc_scatter_add(msg.reshape(NROW, CHUNK, D), dst, zeros)
        st = _tc_node_update(part[0, :N], part[1, :N], d0, d1, st,
                             Wroot, bconv, gru_w)

    bcol = batch.reshape(N, 1)
    o = _tc_set2set(st, st, bcol, fw, Wlin, blin, lstm_w, W1, b1, W2, b2)
    return o.reshape(-1)


# speedup vs baseline: 2.4603x; 2.4603x over previous
import functools

import jax
import jax.numpy as jnp
from jax import lax
from jax.experimental import pallas as pl
from jax.experimental.pallas import tpu as pltpu
from jax.experimental.pallas import tpu_sc as plsc

N, E, F_IN, D, B = 10000, 160000, 128, 32, 256

NC, NS = 2, 16
NW = NC * NS
CHUNK = 128
CH_PER_W = 40
EP = NW * CH_PER_W * CHUNK
NROW = EP // CHUNK
NPAD = 10048
ROWS_PER_SUB = NPAD // NS

_MESH = dict(core_axis_name="c", subcore_axis_name="s", num_cores=NC,
             num_subcores=NS)
_SC_PARAMS = pltpu.CompilerParams(use_tc_tiling_on_sc=False)


def _sc_gather(table, idx2d):
    mesh = plsc.VectorSubcoreMesh(**_MESH)

    @functools.partial(
        pl.kernel,
        out_type=jax.ShapeDtypeStruct((NROW, CHUNK, D), jnp.float32),
        mesh=mesh,
        scratch_types=[pltpu.VMEM((CHUNK,), jnp.int32),
                       pltpu.VMEM((CHUNK, D), jnp.float32),
                       pltpu.SemaphoreType.DMA],
        compiler_params=_SC_PARAMS,
    )
    def k(table_hbm, idx_hbm, out_hbm, idx_v, rows_v, sem):
        wid = lax.axis_index("s") * NC + lax.axis_index("c")

        def body(j, carry):
            r = wid * CH_PER_W + j
            pltpu.sync_copy(idx_hbm.at[r], idx_v)
            pltpu.async_copy(table_hbm.at[idx_v], rows_v, sem).wait()
            pltpu.sync_copy(rows_v, out_hbm.at[r])
            return carry

        lax.fori_loop(0, CH_PER_W, body, 0)

    return k(table, idx2d)


def _sc_scatter_add(val3, idx2d, zeros):
    mesh = plsc.VectorSubcoreMesh(**_MESH)

    @functools.partial(
        pl.kernel,
        out_type=jax.ShapeDtypeStruct((NC, NPAD, D), jnp.float32),
        mesh=mesh,
        scratch_types=[pltpu.VMEM((CHUNK,), jnp.int32),
                       pltpu.VMEM((CHUNK, D), jnp.float32),
                       pltpu.VMEM_SHARED((NPAD, D), jnp.float32)],
        compiler_params=_SC_PARAMS,
    )
    def k(val_hbm, idx_hbm, z_hbm, out_hbm, idx_v, val_v, acc):
        cid = lax.axis_index("c")
        sid = lax.axis_index("s")
        wid = sid * NC + cid
        rows0 = sid * ROWS_PER_SUB
        pltpu.sync_copy(z_hbm.at[pl.ds(rows0, ROWS_PER_SUB)],
                        acc.at[pl.ds(rows0, ROWS_PER_SUB)])
        plsc.subcore_barrier()

        def body(j, carry):
            r = wid * CH_PER_W + j
            pltpu.sync_copy(idx_hbm.at[r], idx_v)
            pltpu.sync_copy(val_hbm.at[r], val_v)
            pltpu.sync_copy(val_v, acc.at[idx_v], add=True)
            return carry

        lax.fori_loop(0, CH_PER_W, body, 0)
        plsc.subcore_barrier()
        pltpu.sync_copy(acc.at[pl.ds(rows0, ROWS_PER_SUB)],
                        out_hbm.at[cid, pl.ds(rows0, ROWS_PER_SUB)])

    return k(val3, idx2d, zeros)


def _tc_node_init(x, W0, b0):
    def body(x_ref, w_ref, b_ref, o_ref):
        o_ref[...] = jnp.maximum(
            jnp.dot(x_ref[...], w_ref[...],
                    preferred_element_type=jnp.float32) + b_ref[...], 0.0)

    return pl.pallas_call(
        body, out_shape=jax.ShapeDtypeStruct((N, D), jnp.float32),
    )(x, W0, b0.reshape(1, D))


_TILE_H1 = 16384


def _tc_edge_mlp1(ea_p, We1p, be1):
    def body(a_ref, w_ref, b_ref, o_ref):
        o_ref[...] = jnp.maximum(
            jnp.dot(a_ref[...], w_ref[...],
                    preferred_element_type=jnp.float32) + b_ref[...], 0.0)

    return pl.pallas_call(
        body,
        grid=(EP // _TILE_H1,),
        in_specs=[pl.BlockSpec((_TILE_H1, 8), lambda i: (i, 0)),
                  pl.BlockSpec((8, 128), lambda i: (0, 0)),
                  pl.BlockSpec((1, 128), lambda i: (0, 0))],
        out_specs=pl.BlockSpec((_TILE_H1, 128), lambda i: (i, 0)),
        out_shape=jax.ShapeDtypeStruct((EP, 128), jnp.float32),
    )(ea_p, We1p, be1.reshape(1, 128))


_TILE_MSG = 2048


def _tc_msg(h1, g, We2, be2, S, R):
    def body(h1_ref, g_ref, w_ref, b_ref, s_ref, r_ref, o_ref):
        h2 = jnp.dot(h1_ref[...], w_ref[...],
                     preferred_element_type=jnp.float32) + b_ref[...]
        g4 = jnp.dot(g_ref[...], s_ref[...],
                     preferred_element_type=jnp.float32)
        o_ref[...] = jnp.dot(h2 * g4, r_ref[...],
                             preferred_element_type=jnp.float32)

    return pl.pallas_call(
        body,
        grid=(EP // _TILE_MSG,),
        in_specs=[pl.BlockSpec((_TILE_MSG, 128), lambda i: (i, 0)),
                  pl.BlockSpec((_TILE_MSG, D), lambda i: (i, 0)),
                  pl.BlockSpec((128, D * D), lambda i: (0, 0)),
                  pl.BlockSpec((1, D * D), lambda i: (0, 0)),
                  pl.BlockSpec((D, D * D), lambda i: (0, 0)),
                  pl.BlockSpec((D * D, D), lambda i: (0, 0))],
        out_specs=pl.BlockSpec((_TILE_MSG, D), lambda i: (i, 0)),
        out_shape=jax.ShapeDtypeStruct((EP, D), jnp.float32),
    )(h1, g, We2, be2.reshape(1, D * D), S, R)


def _tc_node_update(a0, a1, d0, d1, st, Wroot, bconv, gru_w):
    (wr, wz, wn, ur, uz, un, br, bz, bn, cr, cz, cn) = gru_w

    def body(a0_ref, a1_ref, d0_ref, d1_ref, s_ref, wroot_ref, bc_ref,
             wr_ref, wz_ref, wn_ref, ur_ref, uz_ref, un_ref,
             br_ref, bz_ref, bn_ref, cr_ref, cz_ref, cn_ref, o_ref):
        deg = jnp.maximum(d0_ref[...] + d1_ref[...], 1.0)
        s = s_ref[...]
        aggr = (a0_ref[...] + a1_ref[...]) / deg
        m = jnp.maximum(
            aggr + jnp.dot(s, wroot_ref[...],
                           preferred_element_type=jnp.float32) + bc_ref[...],
            0.0)
        mm = lambda a, w: jnp.dot(a, w[...], preferred_element_type=jnp.float32)
        r = jax.nn.sigmoid(mm(m, wr_ref) + br_ref[...]
                           + mm(s, ur_ref) + cr_ref[...])
        z = jax.nn.sigmoid(mm(m, wz_ref) + bz_ref[...]
                           + mm(s, uz_ref) + cz_ref[...])
        n = jnp.tanh(mm(m, wn_ref) + bn_ref[...]
                     + r * (mm(s, un_ref) + cn_ref[...]))
        o_ref[...] = (1.0 - z) * n + z * s

    return pl.pallas_call(
        body, out_shape=jax.ShapeDtypeStruct((N, D), jnp.float32),
    )(a0, a1, d0, d1, st, Wroot, bconv.reshape(1, D),
      wr, wz, wn, ur, uz, un,
      br.reshape(1, D), bz.reshape(1, D), bn.reshape(1, D),
      cr.reshape(1, D), cz.reshape(1, D), cn.reshape(1, D))


def _tc_set2set(conv1, st, bcol, fw, Wlin, blin, lstm_w, W1, b1, W2, b2):
    (wi_i, wi_f, wi_g, wi_o, ui_i, ui_f, ui_g, ui_o,
     bl_i, bl_f, bl_g, bl_o) = lstm_w

    def body(c_ref, h_ref, b_ref, fw_ref, wlin_ref, blin_ref,
             wii_ref, wif_ref, wig_ref, wio_ref,
             uii_ref, uif_ref, uig_ref, uio_ref,
             bli_ref, blf_ref, blg_ref, blo_ref,
             w1_ref, b1_ref, w2_ref, b2_ref, o_ref):
        mm = lambda a, w: jnp.dot(a, w, preferred_element_type=jnp.float32)
        conv1_v = c_ref[...]
        out = h_ref[...]
        fwv = fw_ref[...]
        wlin = wlin_ref[...]
        blin = blin_ref[...]
        for _ in range(3):
            out = out + jnp.maximum(mm(fwv * out, wlin) + blin, 0.0)
        xx = jnp.concatenate([conv1_v, out], axis=1)
        bcol_v = b_ref[...]
        iot = lax.broadcasted_iota(jnp.int32, (N, B), 1)
        oh = bcol_v == iot
        ohf = oh.astype(jnp.float32)
        q_star = jnp.zeros((B, 4 * D), jnp.float32)
        hl = jnp.zeros((B, 2 * D), jnp.float32)
        cl = jnp.zeros((B, 2 * D), jnp.float32)
        for _ in range(3):
            gi = mm(q_star, wii_ref[...]) + mm(hl, uii_ref[...]) + bli_ref[...]
            gf = mm(q_star, wif_ref[...]) + mm(hl, uif_ref[...]) + blf_ref[...]
            gg = mm(q_star, wig_ref[...]) + mm(hl, uig_ref[...]) + blg_ref[...]
            go = mm(q_star, wio_ref[...]) + mm(hl, uio_ref[...]) + blo_ref[...]
            cl = jax.nn.sigmoid(gf) * cl + jax.nn.sigmoid(gi) * jnp.tanh(gg)
            hl = jax.nn.sigmoid(go) * jnp.tanh(cl)
            q = hl
            eq = lax.dot_general(xx, q, (((1,), (1,)), ((), ())))
            e = jnp.sum(eq * ohf, axis=1, keepdims=True)
            masked = jnp.where(oh, e, -1e30)
            mseg = jnp.max(masked, axis=0, keepdims=True)
            mseg = jnp.where(mseg > -1e29, mseg, 0.0)
            ex = jnp.exp(e - jnp.sum(ohf * mseg, axis=1, keepdims=True))
            sseg = jnp.sum(ohf * ex, axis=0, keepdims=True)
            a = ex / (jnp.sum(ohf * sseg, axis=1, keepdims=True) + 1e-16)
            rvec = lax.dot_general(ohf, a * xx,
                                   (((0,), (0,)), ((), ())))
            q_star = jnp.concatenate([q, rvec], axis=1)
        o1 = jnp.maximum(mm(q_star, w1_ref[...]) + b1_ref[...], 0.0)
        o_ref[...] = mm(o1, w2_ref[...]) + b2_ref[...]

    return pl.pallas_call(
        body, out_shape=jax.ShapeDtypeStruct((B, 1), jnp.float32),
    )(conv1, st, bcol, fw, Wlin, blin.reshape(1, D),
      wi_i, wi_f, wi_g, wi_o, ui_i, ui_f, ui_g, ui_o,
      bl_i.reshape(1, 2 * D), bl_f.reshape(1, 2 * D),
      bl_g.reshape(1, 2 * D), bl_o.reshape(1, 2 * D),
      W1, b1.reshape(1, 4 * D), W2, b2.reshape(1, 1))


def kernel(x, edge_index, edge_attr, batch, W0, b0, We1, be1, We2, be2,
           Wroot, bconv, Wih, bih, Whh, bhh, fw, Wlin, blin,
           Wih_l, bih_l, Whh_l, bhh_l, W1, b1, W2, b2):
    src = jnp.pad(edge_index[0], (0, EP - E)).reshape(NROW, CHUNK)
    dst = jnp.pad(edge_index[1], (0, EP - E),
                  constant_values=N).reshape(NROW, CHUNK)
    ea_p = jnp.pad(edge_attr, ((0, EP - E), (0, 3)))
    We1p = jnp.pad(We1, ((0, 3), (0, 0)))
    zeros = jnp.zeros((NPAD, D), jnp.float32)
    ones3 = jnp.ones((NROW, CHUNK, D), jnp.float32)
    eye = jnp.eye(D, dtype=jnp.float32)
    S = jnp.repeat(eye, D, axis=1)
    R = jnp.tile(eye, (D, 1))

    gru_w = (Wih[:, 0:D], Wih[:, D:2 * D], Wih[:, 2 * D:],
             Whh[:, 0:D], Whh[:, D:2 * D], Whh[:, 2 * D:],
             bih[0:D], bih[D:2 * D], bih[2 * D:],
             bhh[0:D], bhh[D:2 * D], bhh[2 * D:])
    Hs = 2 * D
    lstm_w = (Wih_l[:, 0:Hs], Wih_l[:, Hs:2 * Hs],
              Wih_l[:, 2 * Hs:3 * Hs], Wih_l[:, 3 * Hs:],
              Whh_l[:, 0:Hs], Whh_l[:, Hs:2 * Hs],
              Whh_l[:, 2 * Hs:3 * Hs], Whh_l[:, 3 * Hs:],
              bih_l[0:Hs] + bhh_l[0:Hs], bih_l[Hs:2 * Hs] + bhh_l[Hs:2 * Hs],
              bih_l[2 * Hs:3 * Hs] + bhh_l[2 * Hs:3 * Hs],
              bih_l[3 * Hs:] + bhh_l[3 * Hs:])

    st = _tc_node_init(x, W0, b0)
    h1 = _tc_edge_mlp1(ea_p, We1p, be1)

    deg_p = _sc_scatter_add(ones3, dst, zeros)
    d0 = deg_p[0, :N, 0:1]
    d1 = deg_p[1, :N, 0:1]

    for _ in range(3):
        g3 = _sc_gather(st, src)
        msg = _tc_msg(h1, g3.reshape(EP, D), We2, be2, S, R)
        part = _sc_scatter_add(msg.reshape(NROW, CHUNK, D), dst, zeros)
        st = _tc_node_update(part[0, :N], part[1, :N], d0, d1, st,
                             Wroot, bconv, gru_w)

    bcol = batch.reshape(N, 1)
    o = _tc_set2set(st, st, bcol, fw, Wlin, blin, lstm_w, W1, b1, W2, b2)
    return o.reshape(-1)


# speedup vs baseline: 2.7507x; 1.1180x over previous
import functools

import jax
import jax.numpy as jnp
from jax import lax
from jax.experimental import pallas as pl
from jax.experimental.pallas import tpu as pltpu
from jax.experimental.pallas import tpu_sc as plsc

N, E, F_IN, D, B = 10000, 160000, 128, 32, 256

NC, NS = 2, 16
NW = NC * NS
CHUNK = 128
CH_PER_W = 40
EP = NW * CH_PER_W * CHUNK
NPAD = 10048
ROWS_PER_SUB = NPAD // NS

_MESH = dict(core_axis_name="c", subcore_axis_name="s", num_cores=NC,
             num_subcores=NS)
_SC_PARAMS = pltpu.CompilerParams(use_tc_tiling_on_sc=False)


def _sc_gather(table, idx):
    mesh = plsc.VectorSubcoreMesh(**_MESH)

    @functools.partial(
        pl.kernel,
        out_type=jax.ShapeDtypeStruct((EP, D), jnp.float32),
        mesh=mesh,
        scratch_types=[pltpu.VMEM((CHUNK,), jnp.int32),
                       pltpu.VMEM((CHUNK, D), jnp.float32),
                       pltpu.SemaphoreType.DMA],
        compiler_params=_SC_PARAMS,
    )
    def k(table_hbm, idx_hbm, out_hbm, idx_v, rows_v, sem):
        wid = lax.axis_index("s") * NC + lax.axis_index("c")

        def body(j, carry):
            base = (wid * CH_PER_W + j) * CHUNK
            pltpu.sync_copy(idx_hbm.at[pl.ds(base, CHUNK)], idx_v)
            pltpu.async_copy(table_hbm.at[idx_v], rows_v, sem).wait()
            pltpu.sync_copy(rows_v, out_hbm.at[pl.ds(base, CHUNK)])
            return carry

        lax.fori_loop(0, CH_PER_W, body, 0)

    return k(table, idx)


def _sc_scatter_add(val, idx, zeros, with_deg):
    mesh = plsc.VectorSubcoreMesh(**_MESH)
    n_out = 2 if with_deg else 1
    scratch = [pltpu.VMEM((CHUNK,), jnp.int32),
               pltpu.VMEM((CHUNK, D), jnp.float32),
               pltpu.VMEM_SHARED((NPAD, D), jnp.float32)]
    if with_deg:
        scratch += [pltpu.VMEM((CHUNK, D), jnp.float32),
                    pltpu.VMEM_SHARED((NPAD, D), jnp.float32)]

    @functools.partial(
        pl.kernel,
        out_type=tuple(jax.ShapeDtypeStruct((NC, NPAD, D), jnp.float32)
                       for _ in range(n_out)),
        mesh=mesh,
        scratch_types=scratch,
        compiler_params=_SC_PARAMS,
    )
    def k(val_hbm, idx_hbm, z_hbm, ones_hbm, *rest):
        if with_deg:
            out_hbm, dout_hbm, idx_v, val_v, acc, ones_v, dacc = rest
        else:
            out_hbm, idx_v, val_v, acc = rest
        cid = lax.axis_index("c")
        sid = lax.axis_index("s")
        wid = sid * NC + cid
        rows0 = sid * ROWS_PER_SUB
        pltpu.sync_copy(z_hbm.at[pl.ds(rows0, ROWS_PER_SUB)],
                        acc.at[pl.ds(rows0, ROWS_PER_SUB)])
        if with_deg:
            pltpu.sync_copy(z_hbm.at[pl.ds(rows0, ROWS_PER_SUB)],
                            dacc.at[pl.ds(rows0, ROWS_PER_SUB)])
            pltpu.sync_copy(ones_hbm, ones_v)
        plsc.subcore_barrier()

        def body(j, carry):
            base = (wid * CH_PER_W + j) * CHUNK
            pltpu.sync_copy(idx_hbm.at[pl.ds(base, CHUNK)], idx_v)
            pltpu.sync_copy(val_hbm.at[pl.ds(base, CHUNK)], val_v)
            pltpu.sync_copy(val_v, acc.at[idx_v], add=True)
            if with_deg:
                pltpu.sync_copy(ones_v, dacc.at[idx_v], add=True)
            return carry

        lax.fori_loop(0, CH_PER_W, body, 0)
        plsc.subcore_barrier()
        pltpu.sync_copy(acc.at[pl.ds(rows0, ROWS_PER_SUB)],
                        out_hbm.at[cid, pl.ds(rows0, ROWS_PER_SUB)])
        if with_deg:
            pltpu.sync_copy(dacc.at[pl.ds(rows0, ROWS_PER_SUB)],
                            dout_hbm.at[cid, pl.ds(rows0, ROWS_PER_SUB)])

    return k(val, idx, zeros, jnp.ones((CHUNK, D), jnp.float32))


def _tc_node_init(x, W0, b0):
    def body(x_ref, w_ref, b_ref, o_ref):
        o_ref[...] = jnp.maximum(
            jnp.dot(x_ref[...], w_ref[...],
                    preferred_element_type=jnp.float32) + b_ref[...], 0.0)

    return pl.pallas_call(
        body, out_shape=jax.ShapeDtypeStruct((N, D), jnp.float32),
    )(x, W0, b0.reshape(1, D))


_TILE_H2 = 2048


def _tc_edge_h2(ea_p, We1p, be1, We2):
    def body(a_ref, w1_ref, b_ref, w2_ref, o_ref):
        h1 = jnp.maximum(
            jnp.dot(a_ref[...], w1_ref[...],
                    preferred_element_type=jnp.float32) + b_ref[...], 0.0)
        o_ref[...] = jnp.dot(
            h1, w2_ref[...],
            preferred_element_type=jnp.float32).astype(jnp.bfloat16)

    return pl.pallas_call(
        body,
        grid=(EP // _TILE_H2,),
        in_specs=[pl.BlockSpec((_TILE_H2, 8), lambda i: (i, 0)),
                  pl.BlockSpec((8, 128), lambda i: (0, 0)),
                  pl.BlockSpec((1, 128), lambda i: (0, 0)),
                  pl.BlockSpec((128, D * D), lambda i: (0, 0))],
        out_specs=pl.BlockSpec((_TILE_H2, D * D), lambda i: (i, 0)),
        out_shape=jax.ShapeDtypeStruct((EP, D * D), jnp.bfloat16),
    )(ea_p, We1p, be1.reshape(1, 128), We2)


_TILE_MSG = 2048


def _tc_msg(h2, g, Be2r, S, R):
    def body(h2_ref, g_ref, s_ref, r_ref, b2r_ref, o_ref):
        gv = g_ref[...]
        g4 = jnp.dot(gv.astype(jnp.bfloat16), s_ref[...],
                     preferred_element_type=jnp.float32).astype(jnp.bfloat16)
        o_ref[...] = (
            jnp.dot(h2_ref[...] * g4, r_ref[...],
                    preferred_element_type=jnp.float32)
            + jnp.dot(gv, b2r_ref[...], preferred_element_type=jnp.float32))

    return pl.pallas_call(
        body,
        grid=(EP // _TILE_MSG,),
        in_specs=[pl.BlockSpec((_TILE_MSG, D * D), lambda i: (i, 0)),
                  pl.BlockSpec((_TILE_MSG, D), lambda i: (i, 0)),
                  pl.BlockSpec((D, D * D), lambda i: (0, 0)),
                  pl.BlockSpec((D * D, D), lambda i: (0, 0)),
                  pl.BlockSpec((D, D), lambda i: (0, 0))],
        out_specs=pl.BlockSpec((_TILE_MSG, D), lambda i: (i, 0)),
        out_shape=jax.ShapeDtypeStruct((EP, D), jnp.float32),
    )(h2, g, S, R, Be2r)


def _tc_node_update(a0, a1, d0, d1, st, Wroot, bconv, gru_w):
    (wr, wz, wn, ur, uz, un, br, bz, bn, cr, cz, cn) = gru_w

    def body(a0_ref, a1_ref, d0_ref, d1_ref, s_ref, wroot_ref, bc_ref,
             wr_ref, wz_ref, wn_ref, ur_ref, uz_ref, un_ref,
             br_ref, bz_ref, bn_ref, cr_ref, cz_ref, cn_ref, o_ref):
        deg = jnp.maximum(d0_ref[...] + d1_ref[...], 1.0)
        s = s_ref[...]
        aggr = (a0_ref[...] + a1_ref[...]) / deg
        m = jnp.maximum(
            aggr + jnp.dot(s, wroot_ref[...],
                           preferred_element_type=jnp.float32) + bc_ref[...],
            0.0)
        mm = lambda a, w: jnp.dot(a, w[...], preferred_element_type=jnp.float32)
        r = jax.nn.sigmoid(mm(m, wr_ref) + br_ref[...]
                           + mm(s, ur_ref) + cr_ref[...])
        z = jax.nn.sigmoid(mm(m, wz_ref) + bz_ref[...]
                           + mm(s, uz_ref) + cz_ref[...])
        n = jnp.tanh(mm(m, wn_ref) + bn_ref[...]
                     + r * (mm(s, un_ref) + cn_ref[...]))
        o_ref[...] = (1.0 - z) * n + z * s

    return pl.pallas_call(
        body, out_shape=jax.ShapeDtypeStruct((N, D), jnp.float32),
    )(a0, a1, d0, d1, st, Wroot, bconv.reshape(1, D),
      wr, wz, wn, ur, uz, un,
      br.reshape(1, D), bz.reshape(1, D), bn.reshape(1, D),
      cr.reshape(1, D), cz.reshape(1, D), cn.reshape(1, D))


def _tc_set2set(conv1, st, bcol, fw, Wlin, blin, lstm_w, W1, b1, W2, b2):
    (wi_i, wi_f, wi_g, wi_o, ui_i, ui_f, ui_g, ui_o,
     bl_i, bl_f, bl_g, bl_o) = lstm_w

    def body(c_ref, h_ref, b_ref, fw_ref, wlin_ref, blin_ref,
             wii_ref, wif_ref, wig_ref, wio_ref,
             uii_ref, uif_ref, uig_ref, uio_ref,
             bli_ref, blf_ref, blg_ref, blo_ref,
             w1_ref, b1_ref, w2_ref, b2_ref, o_ref):
        mm = lambda a, w: jnp.dot(a, w, preferred_element_type=jnp.float32)
        conv1_v = c_ref[...]
        out = h_ref[...]
        fwv = fw_ref[...]
        wlin = wlin_ref[...]
        blin = blin_ref[...]
        for _ in range(3):
            out = out + jnp.maximum(mm(fwv * out, wlin) + blin, 0.0)
        xx = jnp.concatenate([conv1_v, out], axis=1)
        bcol_v = b_ref[...]
        iot = lax.broadcasted_iota(jnp.int32, (N, B), 1)
        oh = bcol_v == iot
        ohf = oh.astype(jnp.float32)
        q_star = jnp.zeros((B, 4 * D), jnp.float32)
        hl = jnp.zeros((B, 2 * D), jnp.float32)
        cl = jnp.zeros((B, 2 * D), jnp.float32)
        for _ in range(3):
            gi = mm(q_star, wii_ref[...]) + mm(hl, uii_ref[...]) + bli_ref[...]
            gf = mm(q_star, wif_ref[...]) + mm(hl, uif_ref[...]) + blf_ref[...]
            gg = mm(q_star, wig_ref[...]) + mm(hl, uig_ref[...]) + blg_ref[...]
            go = mm(q_star, wio_ref[...]) + mm(hl, uio_ref[...]) + blo_ref[...]
            cl = jax.nn.sigmoid(gf) * cl + jax.nn.sigmoid(gi) * jnp.tanh(gg)
            hl = jax.nn.sigmoid(go) * jnp.tanh(cl)
            q = hl
            eq = lax.dot_general(xx, q, (((1,), (1,)), ((), ())))
            e = jnp.sum(eq * ohf, axis=1, keepdims=True)
            masked = jnp.where(oh, e, -1e30)
            mseg = jnp.max(masked, axis=0, keepdims=True)
            mseg = jnp.where(mseg > -1e29, mseg, 0.0)
            ex = jnp.exp(e - jnp.sum(ohf * mseg, axis=1, keepdims=True))
            sseg = jnp.sum(ohf * ex, axis=0, keepdims=True)
            a = ex / (jnp.sum(ohf * sseg, axis=1, keepdims=True) + 1e-16)
            rvec = lax.dot_general(ohf, a * xx,
                                   (((0,), (0,)), ((), ())))
            q_star = jnp.concatenate([q, rvec], axis=1)
        o1 = jnp.maximum(mm(q_star, w1_ref[...]) + b1_ref[...], 0.0)
        o_ref[...] = mm(o1, w2_ref[...]) + b2_ref[...]

    return pl.pallas_call(
        body, out_shape=jax.ShapeDtypeStruct((B, 1), jnp.float32),
    )(conv1, st, bcol, fw, Wlin, blin.reshape(1, D),
      wi_i, wi_f, wi_g, wi_o, ui_i, ui_f, ui_g, ui_o,
      bl_i.reshape(1, 2 * D), bl_f.reshape(1, 2 * D),
      bl_g.reshape(1, 2 * D), bl_o.reshape(1, 2 * D),
      W1, b1.reshape(1, 4 * D), W2, b2.reshape(1, 1))


def kernel(x, edge_index, edge_attr, batch, W0, b0, We1, be1, We2, be2,
           Wroot, bconv, Wih, bih, Whh, bhh, fw, Wlin, blin,
           Wih_l, bih_l, Whh_l, bhh_l, W1, b1, W2, b2):
    src = jnp.pad(edge_index[0], (0, EP - E))
    dst = jnp.pad(edge_index[1], (0, EP - E), constant_values=N)
    ea_p = jnp.pad(edge_attr, ((0, EP - E), (0, 3)))
    We1p = jnp.pad(We1, ((0, 3), (0, 0)))
    zeros = jnp.zeros((NPAD, D), jnp.float32)
    eye = jnp.eye(D, dtype=jnp.float32)
    S = jnp.repeat(eye, D, axis=1).astype(jnp.bfloat16)
    R = jnp.tile(eye, (D, 1)).astype(jnp.bfloat16)
    Be2r = be2.reshape(D, D)

    gru_w = (Wih[:, 0:D], Wih[:, D:2 * D], Wih[:, 2 * D:],
             Whh[:, 0:D], Whh[:, D:2 * D], Whh[:, 2 * D:],
             bih[0:D], bih[D:2 * D], bih[2 * D:],
             bhh[0:D], bhh[D:2 * D], bhh[2 * D:])
    Hs = 2 * D
    lstm_w = (Wih_l[:, 0:Hs], Wih_l[:, Hs:2 * Hs],
              Wih_l[:, 2 * Hs:3 * Hs], Wih_l[:, 3 * Hs:],
              Whh_l[:, 0:Hs], Whh_l[:, Hs:2 * Hs],
              Whh_l[:, 2 * Hs:3 * Hs], Whh_l[:, 3 * Hs:],
              bih_l[0:Hs] + bhh_l[0:Hs], bih_l[Hs:2 * Hs] + bhh_l[Hs:2 * Hs],
              bih_l[2 * Hs:3 * Hs] + bhh_l[2 * Hs:3 * Hs],
              bih_l[3 * Hs:] + bhh_l[3 * Hs:])

    st = _tc_node_init(x, W0, b0)
    h2 = _tc_edge_h2(ea_p, We1p, be1, We2)

    d0 = d1 = None
    for r in range(3):
        g = _sc_gather(st, src)
        msg = _tc_msg(h2, g, Be2r, S, R)
        if r == 0:
            part, degp = _sc_scatter_add(msg, dst, zeros, True)
            d0 = degp[0, :N, 0:1]
            d1 = degp[1, :N, 0:1]
        else:
            (part,) = _sc_scatter_add(msg, dst, zeros, False)
        st = _tc_node_update(part[0, :N], part[1, :N], d0, d1, st,
                             Wroot, bconv, gru_w)

    bcol = batch.reshape(N, 1)
    o = _tc_set2set(st, st, bcol, fw, Wlin, blin, lstm_w, W1, b1, W2, b2)
    return o.reshape(-1)


# speedup vs baseline: 2.9435x; 1.0701x over previous
import functools

import jax
import jax.numpy as jnp
from jax import lax
from jax.experimental import pallas as pl
from jax.experimental.pallas import tpu as pltpu
from jax.experimental.pallas import tpu_sc as plsc

N, E, F_IN, D, B = 10000, 160000, 128, 32, 256

NC, NS = 2, 16
NW = NC * NS
CHUNK = 128
CH_PER_W = 40
EP = NW * CH_PER_W * CHUNK
NPAD = 10048
ROWS_PER_SUB = NPAD // NS

_MESH = dict(core_axis_name="c", subcore_axis_name="s", num_cores=NC,
             num_subcores=NS)
_SC_PARAMS = pltpu.CompilerParams(use_tc_tiling_on_sc=False)


def _sc_gather(table, idx):
    mesh = plsc.VectorSubcoreMesh(**_MESH)

    @functools.partial(
        pl.kernel,
        out_type=jax.ShapeDtypeStruct((EP, D), jnp.float32),
        mesh=mesh,
        scratch_types=[pltpu.VMEM((CHUNK,), jnp.int32),
                       pltpu.VMEM((CHUNK,), jnp.int32),
                       pltpu.VMEM((CHUNK, D), jnp.float32),
                       pltpu.VMEM((CHUNK, D), jnp.float32),
                       pltpu.SemaphoreType.DMA,
                       pltpu.SemaphoreType.DMA],
        compiler_params=_SC_PARAMS,
    )
    def k(table_hbm, idx_hbm, out_hbm, idx_v0, idx_v1, rows_v0, rows_v1,
          sem0, sem1):
        wid = lax.axis_index("s") * NC + lax.axis_index("c")

        def body(j2, carry):
            b0 = (wid * CH_PER_W + 2 * j2) * CHUNK
            b1 = b0 + CHUNK
            pltpu.sync_copy(idx_hbm.at[pl.ds(b0, CHUNK)], idx_v0)
            cp0 = pltpu.async_copy(table_hbm.at[idx_v0], rows_v0, sem0)
            pltpu.sync_copy(idx_hbm.at[pl.ds(b1, CHUNK)], idx_v1)
            cp1 = pltpu.async_copy(table_hbm.at[idx_v1], rows_v1, sem1)
            cp0.wait()
            pltpu.sync_copy(rows_v0, out_hbm.at[pl.ds(b0, CHUNK)])
            cp1.wait()
            pltpu.sync_copy(rows_v1, out_hbm.at[pl.ds(b1, CHUNK)])
            return carry

        lax.fori_loop(0, CH_PER_W // 2, body, 0)

    return k(table, idx)


def _sc_scatter_add(val, idx, zeros, with_deg):
    mesh = plsc.VectorSubcoreMesh(**_MESH)
    n_out = 2 if with_deg else 1
    scratch = [pltpu.VMEM((CHUNK,), jnp.int32),
               pltpu.VMEM((CHUNK,), jnp.int32),
               pltpu.VMEM((CHUNK, D), jnp.float32),
               pltpu.VMEM((CHUNK, D), jnp.float32),
               pltpu.VMEM_SHARED((NPAD, D), jnp.float32),
               pltpu.SemaphoreType.DMA,
               pltpu.SemaphoreType.DMA]
    if with_deg:
        scratch += [pltpu.VMEM((CHUNK, D), jnp.float32),
                    pltpu.VMEM_SHARED((NPAD, D), jnp.float32)]

    @functools.partial(
        pl.kernel,
        out_type=tuple(jax.ShapeDtypeStruct((NC, NPAD, D), jnp.float32)
                       for _ in range(n_out)),
        mesh=mesh,
        scratch_types=scratch,
        compiler_params=_SC_PARAMS,
    )
    def k(val_hbm, idx_hbm, z_hbm, ones_hbm, *rest):
        if with_deg:
            (out_hbm, dout_hbm, idx_v0, idx_v1, val_v0, val_v1, acc,
             sem0, sem1, ones_v, dacc) = rest
        else:
            (out_hbm, idx_v0, idx_v1, val_v0, val_v1, acc,
             sem0, sem1) = rest
        cid = lax.axis_index("c")
        sid = lax.axis_index("s")
        wid = sid * NC + cid
        rows0 = sid * ROWS_PER_SUB
        pltpu.sync_copy(z_hbm.at[pl.ds(rows0, ROWS_PER_SUB)],
                        acc.at[pl.ds(rows0, ROWS_PER_SUB)])
        if with_deg:
            pltpu.sync_copy(z_hbm.at[pl.ds(rows0, ROWS_PER_SUB)],
                            dacc.at[pl.ds(rows0, ROWS_PER_SUB)])
            pltpu.sync_copy(ones_hbm, ones_v)
        plsc.subcore_barrier()

        def body(j2, carry):
            b0 = (wid * CH_PER_W + 2 * j2) * CHUNK
            b1 = b0 + CHUNK
            c0a = pltpu.async_copy(idx_hbm.at[pl.ds(b0, CHUNK)], idx_v0, sem0)
            c0b = pltpu.async_copy(val_hbm.at[pl.ds(b0, CHUNK)], val_v0, sem0)
            c1a = pltpu.async_copy(idx_hbm.at[pl.ds(b1, CHUNK)], idx_v1, sem1)
            c1b = pltpu.async_copy(val_hbm.at[pl.ds(b1, CHUNK)], val_v1, sem1)
            c0a.wait()
            c0b.wait()
            pltpu.sync_copy(val_v0, acc.at[idx_v0], add=True)
            if with_deg:
                pltpu.sync_copy(ones_v, dacc.at[idx_v0], add=True)
            c1a.wait()
            c1b.wait()
            pltpu.sync_copy(val_v1, acc.at[idx_v1], add=True)
            if with_deg:
                pltpu.sync_copy(ones_v, dacc.at[idx_v1], add=True)
            return carry

        lax.fori_loop(0, CH_PER_W // 2, body, 0)
        plsc.subcore_barrier()
        pltpu.sync_copy(acc.at[pl.ds(rows0, ROWS_PER_SUB)],
                        out_hbm.at[cid, pl.ds(rows0, ROWS_PER_SUB)])
        if with_deg:
            pltpu.sync_copy(dacc.at[pl.ds(rows0, ROWS_PER_SUB)],
                            dout_hbm.at[cid, pl.ds(rows0, ROWS_PER_SUB)])

    return k(val, idx, zeros, jnp.ones((CHUNK, D), jnp.float32))


def _tc_node_init(x, W0, b0):
    def body(x_ref, w_ref, b_ref, o_ref):
        o_ref[...] = jnp.maximum(
            jnp.dot(x_ref[...], w_ref[...],
                    preferred_element_type=jnp.float32) + b_ref[...], 0.0)

    return pl.pallas_call(
        body, out_shape=jax.ShapeDtypeStruct((N, D), jnp.float32),
    )(x, W0, b0.reshape(1, D))


_TILE_H2 = 2048


def _tc_edge_h2(ea_p, We1p, be1, We2):
    def body(a_ref, w1_ref, b_ref, w2_ref, o_ref):
        h1 = jnp.maximum(
            jnp.dot(a_ref[...], w1_ref[...],
                    preferred_element_type=jnp.float32) + b_ref[...], 0.0)
        o_ref[...] = jnp.dot(
            h1, w2_ref[...],
            preferred_element_type=jnp.float32).astype(jnp.bfloat16)

    return pl.pallas_call(
        body,
        grid=(EP // _TILE_H2,),
        in_specs=[pl.BlockSpec((_TILE_H2, 8), lambda i: (i, 0)),
                  pl.BlockSpec((8, 128), lambda i: (0, 0)),
                  pl.BlockSpec((1, 128), lambda i: (0, 0)),
                  pl.BlockSpec((128, D * D), lambda i: (0, 0))],
        out_specs=pl.BlockSpec((_TILE_H2, D * D), lambda i: (i, 0)),
        out_shape=jax.ShapeDtypeStruct((EP, D * D), jnp.bfloat16),
    )(ea_p, We1p, be1.reshape(1, 128), We2)


_TILE_MSG = 2048


def _tc_msg(h2, g, Be2r, S, R):
    def body(h2_ref, g_ref, s_ref, r_ref, b2r_ref, o_ref):
        gv = g_ref[...]
        g4 = jnp.dot(gv.astype(jnp.bfloat16), s_ref[...],
                     preferred_element_type=jnp.float32).astype(jnp.bfloat16)
        o_ref[...] = (
            jnp.dot(h2_ref[...] * g4, r_ref[...],
                    preferred_element_type=jnp.float32)
            + jnp.dot(gv, b2r_ref[...], preferred_element_type=jnp.float32))

    return pl.pallas_call(
        body,
        grid=(EP // _TILE_MSG,),
        in_specs=[pl.BlockSpec((_TILE_MSG, D * D), lambda i: (i, 0)),
                  pl.BlockSpec((_TILE_MSG, D), lambda i: (i, 0)),
                  pl.BlockSpec((D, D * D), lambda i: (0, 0)),
                  pl.BlockSpec((D * D, D), lambda i: (0, 0)),
                  pl.BlockSpec((D, D), lambda i: (0, 0))],
        out_specs=pl.BlockSpec((_TILE_MSG, D), lambda i: (i, 0)),
        out_shape=jax.ShapeDtypeStruct((EP, D), jnp.float32),
    )(h2, g, S, R, Be2r)


def _tc_node_update(a0, a1, d0, d1, st, Wroot, bconv, gru_w):
    (wr, wz, wn, ur, uz, un, br, bz, bn, cr, cz, cn) = gru_w

    def body(a0_ref, a1_ref, d0_ref, d1_ref, s_ref, wroot_ref, bc_ref,
             wr_ref, wz_ref, wn_ref, ur_ref, uz_ref, un_ref,
             br_ref, bz_ref, bn_ref, cr_ref, cz_ref, cn_ref, o_ref):
        deg = jnp.maximum(d0_ref[...] + d1_ref[...], 1.0)
        s = s_ref[...]
        aggr = (a0_ref[...] + a1_ref[...]) / deg
        m = jnp.maximum(
            aggr + jnp.dot(s, wroot_ref[...],
                           preferred_element_type=jnp.float32) + bc_ref[...],
            0.0)
        mm = lambda a, w: jnp.dot(a, w[...], preferred_element_type=jnp.float32)
        r = jax.nn.sigmoid(mm(m, wr_ref) + br_ref[...]
                           + mm(s, ur_ref) + cr_ref[...])
        z = jax.nn.sigmoid(mm(m, wz_ref) + bz_ref[...]
                           + mm(s, uz_ref) + cz_ref[...])
        n = jnp.tanh(mm(m, wn_ref) + bn_ref[...]
                     + r * (mm(s, un_ref) + cn_ref[...]))
        o_ref[...] = (1.0 - z) * n + z * s

    return pl.pallas_call(
        body, out_shape=jax.ShapeDtypeStruct((N, D), jnp.float32),
    )(a0, a1, d0, d1, st, Wroot, bconv.reshape(1, D),
      wr, wz, wn, ur, uz, un,
      br.reshape(1, D), bz.reshape(1, D), bn.reshape(1, D),
      cr.reshape(1, D), cz.reshape(1, D), cn.reshape(1, D))


def _tc_set2set(conv1, st, bcol, fw, Wlin, blin, lstm_w, W1, b1, W2, b2):
    (wi_i, wi_f, wi_g, wi_o, ui_i, ui_f, ui_g, ui_o,
     bl_i, bl_f, bl_g, bl_o) = lstm_w

    def body(c_ref, h_ref, b_ref, fw_ref, wlin_ref, blin_ref,
             wii_ref, wif_ref, wig_ref, wio_ref,
             uii_ref, uif_ref, uig_ref, uio_ref,
             bli_ref, blf_ref, blg_ref, blo_ref,
             w1_ref, b1_ref, w2_ref, b2_ref, o_ref):
        mm = lambda a, w: jnp.dot(a, w, preferred_element_type=jnp.float32)
        conv1_v = c_ref[...]
        out = h_ref[...]
        fwv = fw_ref[...]
        wlin = wlin_ref[...]
        blin = blin_ref[...]
        for _ in range(3):
            out = out + jnp.maximum(mm(fwv * out, wlin) + blin, 0.0)
        xx = jnp.concatenate([conv1_v, out], axis=1)
        bcol_v = b_ref[...]
        iot = lax.broadcasted_iota(jnp.int32, (N, B), 1)
        oh = bcol_v == iot
        ohf = oh.astype(jnp.float32)
        q_star = jnp.zeros((B, 4 * D), jnp.float32)
        hl = jnp.zeros((B, 2 * D), jnp.float32)
        cl = jnp.zeros((B, 2 * D), jnp.float32)
        for _ in range(3):
            gi = mm(q_star, wii_ref[...]) + mm(hl, uii_ref[...]) + bli_ref[...]
            gf = mm(q_star, wif_ref[...]) + mm(hl, uif_ref[...]) + blf_ref[...]
            gg = mm(q_star, wig_ref[...]) + mm(hl, uig_ref[...]) + blg_ref[...]
            go = mm(q_star, wio_ref[...]) + mm(hl, uio_ref[...]) + blo_ref[...]
            cl = jax.nn.sigmoid(gf) * cl + jax.nn.sigmoid(gi) * jnp.tanh(gg)
            hl = jax.nn.sigmoid(go) * jnp.tanh(cl)
            q = hl
            eq = lax.dot_general(xx, q, (((1,), (1,)), ((), ())))
            e = jnp.sum(eq * ohf, axis=1, keepdims=True)
            masked = jnp.where(oh, e, -1e30)
            mseg = jnp.max(masked, axis=0, keepdims=True)
            mseg = jnp.where(mseg > -1e29, mseg, 0.0)
            ex = jnp.exp(e - jnp.sum(ohf * mseg, axis=1, keepdims=True))
            sseg = jnp.sum(ohf * ex, axis=0, keepdims=True)
            a = ex / (jnp.sum(ohf * sseg, axis=1, keepdims=True) + 1e-16)
            rvec = lax.dot_general(ohf, a * xx,
                                   (((0,), (0,)), ((), ())))
            q_star = jnp.concatenate([q, rvec], axis=1)
        o1 = jnp.maximum(mm(q_star, w1_ref[...]) + b1_ref[...], 0.0)
        o_ref[...] = mm(o1, w2_ref[...]) + b2_ref[...]

    return pl.pallas_call(
        body, out_shape=jax.ShapeDtypeStruct((B, 1), jnp.float32),
    )(conv1, st, bcol, fw, Wlin, blin.reshape(1, D),
      wi_i, wi_f, wi_g, wi_o, ui_i, ui_f, ui_g, ui_o,
      bl_i.reshape(1, 2 * D), bl_f.reshape(1, 2 * D),
      bl_g.reshape(1, 2 * D), bl_o.reshape(1, 2 * D),
      W1, b1.reshape(1, 4 * D), W2, b2.reshape(1, 1))


def kernel(x, edge_index, edge_attr, batch, W0, b0, We1, be1, We2, be2,
           Wroot, bconv, Wih, bih, Whh, bhh, fw, Wlin, blin,
           Wih_l, bih_l, Whh_l, bhh_l, W1, b1, W2, b2):
    src = jnp.pad(edge_index[0], (0, EP - E))
    dst = jnp.pad(edge_index[1], (0, EP - E), constant_values=N)
    ea_p = jnp.pad(edge_attr, ((0, EP - E), (0, 3)))
    We1p = jnp.pad(We1, ((0, 3), (0, 0)))
    zeros = jnp.zeros((NPAD, D), jnp.float32)
    eye = jnp.eye(D, dtype=jnp.float32)
    S = jnp.repeat(eye, D, axis=1).astype(jnp.bfloat16)
    R = jnp.tile(eye, (D, 1)).astype(jnp.bfloat16)
    Be2r = be2.reshape(D, D)

    gru_w = (Wih[:, 0:D], Wih[:, D:2 * D], Wih[:, 2 * D:],
             Whh[:, 0:D], Whh[:, D:2 * D], Whh[:, 2 * D:],
             bih[0:D], bih[D:2 * D], bih[2 * D:],
             bhh[0:D], bhh[D:2 * D], bhh[2 * D:])
    Hs = 2 * D
    lstm_w = (Wih_l[:, 0:Hs], Wih_l[:, Hs:2 * Hs],
              Wih_l[:, 2 * Hs:3 * Hs], Wih_l[:, 3 * Hs:],
              Whh_l[:, 0:Hs], Whh_l[:, Hs:2 * Hs],
              Whh_l[:, 2 * Hs:3 * Hs], Whh_l[:, 3 * Hs:],
              bih_l[0:Hs] + bhh_l[0:Hs], bih_l[Hs:2 * Hs] + bhh_l[Hs:2 * Hs],
              bih_l[2 * Hs:3 * Hs] + bhh_l[2 * Hs:3 * Hs],
              bih_l[3 * Hs:] + bhh_l[3 * Hs:])

    st = _tc_node_init(x, W0, b0)
    h2 = _tc_edge_h2(ea_p, We1p, be1, We2)

    d0 = d1 = None
    for r in range(3):
        g = _sc_gather(st, src)
        msg = _tc_msg(h2, g, Be2r, S, R)
        if r == 0:
            part, degp = _sc_scatter_add(msg, dst, zeros, True)
            d0 = degp[0, :N, 0:1]
            d1 = degp[1, :N, 0:1]
        else:
            (part,) = _sc_scatter_add(msg, dst, zeros, False)
        st = _tc_node_update(part[0, :N], part[1, :N], d0, d1, st,
                             Wroot, bconv, gru_w)

    bcol = batch.reshape(N, 1)
    o = _tc_set2set(st, st, bcol, fw, Wlin, blin, lstm_w, W1, b1, W2, b2)
    return o.reshape(-1)
